# baseline (speedup 1.0000x reference)
import functools

import jax
import jax.numpy as jnp
import numpy as np
from jax.experimental import pallas as pl
from jax.experimental.pallas import tpu as pltpu

_STD = (0.1, 0.1, 0.2, 0.2)
_NMS_THR = 0.7
_N = 20000
_NP = 20480
_NB = 10
_BR = 16
_K = 2048
_KN = 2000
_KO = 1000

def _ploc():
    r = jax.lax.broadcasted_iota(jnp.int32, (1, _BR, 128), 1)
    l = jax.lax.broadcasted_iota(jnp.int32, (1, _BR, 128), 2)
    return 128 * r + l


def _hi_mask(j):
    return (_ploc() & j) != 0


def _asc_mask(k, flip):
    m = (_ploc() & k) != 0
    return ~m if flip else m


def _roll_lanes(x, j):
    return jnp.concatenate([x[..., j:], x[..., :j]], axis=-1)


def _lane_partner(x, j):
    lo = _roll_lanes(x, j)
    hi = _roll_lanes(x, 128 - j)
    return jnp.where(_hi_mask(j), hi, lo)


def _row_partner(x, jrow):
    nb = x.shape[0]
    g = _BR // (2 * jrow)
    xr = x.reshape(nb, g, 2, jrow, 128)
    sw = jnp.concatenate([xr[:, :, 1:2], xr[:, :, 0:1]], axis=2)
    return sw.reshape(nb, _BR, 128)


def _cmpex(key, tie, pay, partner_fn, hi, asc):
    kp = partner_fn(key)
    if tie is None:
        better = key > kp
        tp = None
    else:
        tp = partner_fn(tie)
        better = (key > kp) | ((key == kp) & (tie < tp))
    keep_self = better ^ hi ^ asc
    nkey = jnp.where(keep_self, key, kp)
    ntie = None if tie is None else jnp.where(keep_self, tie, tp)
    npay = [jnp.where(keep_self, a, partner_fn(a)) for a in pay]
    return nkey, ntie, npay


def _stage(key, tie, pay, j, asc):
    if j >= 128:
        pf = functools.partial(_row_partner, jrow=j // 128)
        hi = _hi_mask(j)
    else:
        pf = functools.partial(_lane_partner, j=j)
        hi = _hi_mask(j)
    return _cmpex(key, tie, pay, pf, hi, asc)


def _sort_blocks(key, tie, pay, ascending=False):
    k = 2
    while k <= _K:
        asc = _asc_mask(k, flip=ascending)
        j = k // 2
        while j >= 1:
            key, tie, pay = _stage(key, tie, pay, j, asc)
            j //= 2
        k *= 2
    return key, tie, pay


def _bitonic_merge(key, tie, pay):
    asc = jnp.zeros((1, _BR, 128), dtype=bool)
    j = _K // 2
    while j >= 1:
        key, tie, pay = _stage(key, tie, pay, j, asc)
        j //= 2
    return key, tie, pay


def _rev_block(x):
    xr = jnp.concatenate([x[:, r:r + 1] for r in range(_BR - 1, -1, -1)],
                         axis=1)
    for j in (64, 32, 16, 8, 4, 2, 1):
        xr = _lane_partner(xr, j)
    return xr


def _merge_level(key, tie, pay):
    nb = key.shape[0]
    m = nb // 2
    take = lambda x, i: x[:2 * m].reshape(m, 2, _BR, 128)[:, i]
    ka, kb = take(key, 0), _rev_block(take(key, 1))
    ta, tb = take(tie, 0), _rev_block(take(tie, 1))
    better = (ka > kb) | ((ka == kb) & (ta < tb))
    wk = jnp.where(better, ka, kb)
    wt = jnp.where(better, ta, tb)
    wp = [jnp.where(better, take(a, 0), _rev_block(take(a, 1))) for a in pay]
    wk, wt, wp = _bitonic_merge(wk, wt, wp)
    if nb % 2:
        wk = jnp.concatenate([wk, key[2 * m:]], axis=0)
        wt = jnp.concatenate([wt, tie[2 * m:]], axis=0)
        wp = [jnp.concatenate([a, b[2 * m:]], axis=0) for a, b in zip(wp, pay)]
    return wk, wt, wp


def _eye128():
    a = jax.lax.broadcasted_iota(jnp.int32, (128, 128), 0)
    b = jax.lax.broadcasted_iota(jnp.int32, (128, 128), 1)
    return (a == b).astype(jnp.float32)


def _row_to_col(v):
    return jax.lax.dot_general(_eye128(), v,
                               (((1,), (1,)), ((), ())),
                               preferred_element_type=jnp.float32)


def _body(fg_ref, d_ref, a_ref, out_ref):
    dy = d_ref[0, 0] * _STD[0]
    dx = d_ref[0, 1] * _STD[1]
    dh = d_ref[0, 2] * _STD[2]
    dw = d_ref[0, 3] * _STD[3]
    ay1, ax1, ay2, ax2 = (a_ref[0, c] for c in range(4))
    height = ay2 - ay1
    width = ax2 - ax1
    center_y = ay1 + 0.5 * height
    center_x = ax1 + 0.5 * width
    center_y = center_y + dy * height
    center_x = center_x + dx * width
    height = height * jnp.exp(dh)
    width = width * jnp.exp(dw)
    y1 = center_y - 0.5 * height
    x1 = center_x - 0.5 * width
    y2 = y1 + height
    x2 = x1 + width
    y1 = jnp.clip(y1, 0.0, 1.0)
    x1 = jnp.clip(x1, 0.0, 1.0)
    y2 = jnp.clip(y2, 0.0, 1.0)
    x2 = jnp.clip(x2, 0.0, 1.0)

    s = fg_ref[0]
    gidx = (128 * jax.lax.broadcasted_iota(jnp.int32, (_NP // 128, 128), 0)
            + jax.lax.broadcasted_iota(jnp.int32, (_NP // 128, 128), 1)
            ).astype(jnp.float32)

    blk = lambda x: x.reshape(_NB, _BR, 128)
    key, tie, pay = _sort_blocks(blk(s), blk(gidx),
                                 [blk(y1), blk(x1), blk(y2), blk(x2)])
    while key.shape[0] > 1:
        key, tie, pay = _merge_level(key, tie, pay)
    y1, x1, y2, x2 = (a[0] for a in pay)

    area = (y2 - y1) * (x2 - x1)
    cols = [jnp.concatenate([_row_to_col(c[r:r + 1]) for r in range(_BR)],
                            axis=0) for c in (y1, x1, y2, x2, area)]
    y1c, x1c, y2c, x2c, arc = cols

    sub_i = jax.lax.broadcasted_iota(jnp.int32, (128, 128), 0)
    lane_i = jax.lax.broadcasted_iota(jnp.int32, (128, 128), 1)
    tri = (sub_i < lane_i).astype(jnp.float32)

    kept_rows = []
    kept_cols = []
    for j in range(_BR):
        p0 = 128 * j
        by1, bx1, by2, bx2 = (c[j:j + 1] for c in (y1, x1, y2, x2))
        bar = area[j:j + 1]
        if j == 0:
            sup = jnp.zeros((1, 128), dtype=bool)
        else:
            res = jnp.concatenate(kept_cols, axis=0)
            yy1 = jnp.maximum(y1c[:p0], by1)
            xx1 = jnp.maximum(x1c[:p0], bx1)
            yy2 = jnp.minimum(y2c[:p0], by2)
            xx2 = jnp.minimum(x2c[:p0], bx2)
            inter = jnp.maximum(yy2 - yy1, 0.0) * jnp.maximum(xx2 - xx1, 0.0)
            union = arc[:p0] + bar - inter
            iou = inter / jnp.maximum(union, 1e-8)
            hit = jnp.where((iou > _NMS_THR) & (res > 0.5), 1.0, 0.0)
            sup = jnp.max(hit, axis=0, keepdims=True) > 0.5
        if p0 + 128 <= _KN:
            valid = jnp.ones((1, 128), dtype=bool)
        else:
            nv = max(_KN - p0, 0)
            valid = jax.lax.broadcasted_iota(jnp.int32, (1, 128), 1) < nv
        cand = (valid & ~sup).astype(jnp.float32)

        yy1 = jnp.maximum(y1c[p0:p0 + 128], by1)
        xx1 = jnp.maximum(x1c[p0:p0 + 128], bx1)
        yy2 = jnp.minimum(y2c[p0:p0 + 128], by2)
        xx2 = jnp.minimum(x2c[p0:p0 + 128], bx2)
        inter = jnp.maximum(yy2 - yy1, 0.0) * jnp.maximum(xx2 - xx1, 0.0)
        union = arc[p0:p0 + 128] + bar - inter
        iou = inter / jnp.maximum(union, 1e-8)
        mat = jnp.where(iou > _NMS_THR, 1.0, 0.0) * tri

        cand8 = jnp.broadcast_to(cand, (8, 128)) + mat[0:8] * 0.0

        def fix_body(carry):
            _, alive, it = carry
            hits = jax.lax.dot_general(alive, mat,
                                       (((1,), (0,)), ((), ())),
                                       preferred_element_type=jnp.float32)
            new = cand8 * jnp.where(hits < 0.5, 1.0, 0.0)
            return alive, new, it + 1

        def fix_cond(carry):
            prev, alive, it = carry
            changed = jnp.sum(jnp.abs(alive - prev)) > 0.0
            return changed & (it < 129)

        _, alive8, _ = jax.lax.while_loop(
            fix_cond, fix_body, (cand8 - 2.0, cand8, jnp.int32(0)))
        alive = alive8[0:1]
        kept_rows.append(alive)
        kept_cols.append(_row_to_col(alive))

    keep = jnp.concatenate(kept_rows, axis=0)
    kept_total = jnp.sum(keep)

    pos = _ploc()[0].astype(jnp.float32)
    ckey = jnp.where(keep > 0.5, pos, pos + float(_K))
    ckey, _, cpay = _sort_blocks(ckey[None], None,
                                 [y1[None], x1[None], y2[None], x2[None]],
                                 ascending=True)
    oy1, ox1, oy2, ox2 = (a[0][:8] for a in cpay)
    slot = _ploc()[0, :8].astype(jnp.float32)
    ok = (slot < jnp.minimum(kept_total, float(_KO))).astype(jnp.float32)
    out_ref[0, 0] = oy1 * ok
    out_ref[0, 1] = ox1 * ok
    out_ref[0, 2] = oy2 * ok
    out_ref[0, 3] = ox2 * ok


def kernel(scores, deltas, anchors):
    b, n, _ = scores.shape
    pad = _NP - n
    fg = jnp.pad(scores[:, :, 1], ((0, 0), (0, pad)), constant_values=-1.0)
    fg = fg.reshape(b, _NP // 128, 128)
    dpk = jnp.pad(jnp.transpose(deltas, (0, 2, 1)), ((0, 0), (0, 0), (0, pad)))
    dpk = dpk.reshape(b, 4, _NP // 128, 128)
    apk = jnp.pad(jnp.transpose(anchors, (0, 2, 1)), ((0, 0), (0, 0), (0, pad)))
    apk = apk.reshape(b, 4, _NP // 128, 128)

    out = pl.pallas_call(
        _body,
        grid=(b,),
        in_specs=[
            pl.BlockSpec((1, _NP // 128, 128), lambda i: (i, 0, 0)),
            pl.BlockSpec((1, 4, _NP // 128, 128), lambda i: (i, 0, 0, 0)),
            pl.BlockSpec((1, 4, _NP // 128, 128), lambda i: (i, 0, 0, 0)),
        ],
        out_specs=pl.BlockSpec((1, 4, 8, 128), lambda i: (i, 0, 0, 0)),
        out_shape=jax.ShapeDtypeStruct((b, 4, 8, 128), jnp.float32),
    )(fg, dpk, apk)
    props = jnp.transpose(out.reshape(b, 4, 1024), (0, 2, 1))[:, :_KO, :]
    return props


# speedup vs baseline: 1.2238x; 1.2238x over previous
import functools

import jax
import jax.numpy as jnp
from jax.experimental import pallas as pl
from jax.experimental.pallas import tpu as pltpu

_STD = (0.1, 0.1, 0.2, 0.2)
_NMS_THR = 0.7
_B = 4
_N = 20000
_NP = 20480
_NB = 10
_BR = 16
_K = 2048
_KN = 2000
_KO = 1000


def _ploc():
    r = jax.lax.broadcasted_iota(jnp.int32, (1, _BR, 128), 1)
    l = jax.lax.broadcasted_iota(jnp.int32, (1, _BR, 128), 2)
    return 128 * r + l


def _hi_mask(j):
    return (_ploc() & j) != 0


def _asc_mask(k, flip):
    m = (_ploc() & k) != 0
    return ~m if flip else m


def _lane_partner(x, j):
    lo = pltpu.roll(x, (-j) % 128, x.ndim - 1)
    hi = pltpu.roll(x, j, x.ndim - 1)
    return jnp.where(_hi_mask(j), hi, lo)


def _row_partner(x, jrow):
    lead = x.shape[:-2]
    g = _BR // (2 * jrow)
    xr = x.reshape(*lead, g, 2, jrow, 128)
    sw = jnp.concatenate([xr[..., 1:2, :, :], xr[..., 0:1, :, :]], axis=-3)
    return sw.reshape(*lead, _BR, 128)


def _cmpex(key, tie, pay, partner_fn, hi, asc):
    kp = partner_fn(key)
    if tie is None:
        better = key > kp
        tp = None
    else:
        tp = partner_fn(tie)
        better = (key > kp) | ((key == kp) & (tie < tp))
    keep_self = better ^ hi ^ asc
    nkey = jnp.where(keep_self, key, kp)
    ntie = None if tie is None else jnp.where(keep_self, tie, tp)
    npay = [jnp.where(keep_self, a, partner_fn(a)) for a in pay]
    return nkey, ntie, npay


def _stage(key, tie, pay, j, asc):
    if j >= 128:
        pf = functools.partial(_row_partner, jrow=j // 128)
    else:
        pf = functools.partial(_lane_partner, j=j)
    return _cmpex(key, tie, pay, pf, _hi_mask(j), asc)


def _sort_blocks(key, tie, pay, ascending=False):
    k = 2
    while k <= _K:
        asc = _asc_mask(k, flip=ascending)
        j = k // 2
        while j >= 1:
            key, tie, pay = _stage(key, tie, pay, j, asc)
            j //= 2
        k *= 2
    return key, tie, pay


def _bitonic_merge(key, tie, pay):
    asc = jnp.zeros((1, _BR, 128), dtype=bool)
    j = _K // 2
    while j >= 1:
        key, tie, pay = _stage(key, tie, pay, j, asc)
        j //= 2
    return key, tie, pay


def _rev_block(x):
    xr = jnp.concatenate([x[..., r:r + 1, :] for r in range(_BR - 1, -1, -1)],
                         axis=-2)
    for j in (64, 32, 16, 8, 4, 2, 1):
        xr = _lane_partner(xr, j)
    return xr


def _merge_level(key, tie, pay):
    nb = key.shape[1]
    m = nb // 2
    take = lambda x, i: x[:, :2 * m].reshape(_B, m, 2, _BR, 128)[:, :, i]
    ka, kb = take(key, 0), _rev_block(take(key, 1))
    ta, tb = take(tie, 0), _rev_block(take(tie, 1))
    better = (ka > kb) | ((ka == kb) & (ta < tb))
    wk = jnp.where(better, ka, kb)
    wt = jnp.where(better, ta, tb)
    wp = [jnp.where(better, take(a, 0), _rev_block(take(a, 1))) for a in pay]
    fl = lambda x: x.reshape(_B * m, _BR, 128)
    ufl = lambda x: x.reshape(_B, m, _BR, 128)
    wk, wt, wp = _bitonic_merge(fl(wk), fl(wt), [fl(a) for a in wp])
    wk, wt, wp = ufl(wk), ufl(wt), [ufl(a) for a in wp]
    if nb % 2:
        wk = jnp.concatenate([wk, key[:, 2 * m:]], axis=1)
        wt = jnp.concatenate([wt, tie[:, 2 * m:]], axis=1)
        wp = [jnp.concatenate([a, b[:, 2 * m:]], axis=1)
              for a, b in zip(wp, pay)]
    return wk, wt, wp


def _eye128():
    a = jax.lax.broadcasted_iota(jnp.int32, (128, 128), 0)
    b = jax.lax.broadcasted_iota(jnp.int32, (128, 128), 1)
    return (a == b).astype(jnp.float32)


def _row_to_col(v):
    return jax.lax.dot_general(_eye128(), v,
                               (((1,), (1,)), ((), ())),
                               preferred_element_type=jnp.float32)


def _nms_keep(y1, x1, y2, x2):
    area = (y2 - y1) * (x2 - x1)
    cols = [jnp.concatenate([_row_to_col(c[r:r + 1]) for r in range(_BR)],
                            axis=0) for c in (y1, x1, y2, x2, area)]
    y1c, x1c, y2c, x2c, arc = cols

    sub_i = jax.lax.broadcasted_iota(jnp.int32, (128, 128), 0)
    lane_i = jax.lax.broadcasted_iota(jnp.int32, (128, 128), 1)
    tri = (sub_i < lane_i).astype(jnp.float32)

    kept_rows = []
    kept_cols = []
    for j in range(_BR):
        p0 = 128 * j
        by1, bx1, by2, bx2 = (c[j:j + 1] for c in (y1, x1, y2, x2))
        bar = area[j:j + 1]
        if j == 0:
            sup = jnp.zeros((1, 128), dtype=bool)
        else:
            res = jnp.concatenate(kept_cols, axis=0)
            yy1 = jnp.maximum(y1c[:p0], by1)
            xx1 = jnp.maximum(x1c[:p0], bx1)
            yy2 = jnp.minimum(y2c[:p0], by2)
            xx2 = jnp.minimum(x2c[:p0], bx2)
            inter = jnp.maximum(yy2 - yy1, 0.0) * jnp.maximum(xx2 - xx1, 0.0)
            union = arc[:p0] + bar - inter
            iou = inter / jnp.maximum(union, 1e-8)
            hit = jnp.where((iou > _NMS_THR) & (res > 0.5), 1.0, 0.0)
            sup = jnp.max(hit, axis=0, keepdims=True) > 0.5
        if p0 + 128 <= _KN:
            valid = jnp.ones((1, 128), dtype=bool)
        else:
            nv = max(_KN - p0, 0)
            valid = jax.lax.broadcasted_iota(jnp.int32, (1, 128), 1) < nv
        cand = (valid & ~sup).astype(jnp.float32)

        yy1 = jnp.maximum(y1c[p0:p0 + 128], by1)
        xx1 = jnp.maximum(x1c[p0:p0 + 128], bx1)
        yy2 = jnp.minimum(y2c[p0:p0 + 128], by2)
        xx2 = jnp.minimum(x2c[p0:p0 + 128], bx2)
        inter = jnp.maximum(yy2 - yy1, 0.0) * jnp.maximum(xx2 - xx1, 0.0)
        union = arc[p0:p0 + 128] + bar - inter
        iou = inter / jnp.maximum(union, 1e-8)
        mat = jnp.where(iou > _NMS_THR, 1.0, 0.0) * tri

        cand8 = jnp.broadcast_to(cand, (8, 128)) + mat[0:8] * 0.0

        def fix_body(carry):
            _, alive, it = carry
            hits = jax.lax.dot_general(alive, mat,
                                       (((1,), (0,)), ((), ())),
                                       preferred_element_type=jnp.float32)
            new = cand8 * jnp.where(hits < 0.5, 1.0, 0.0)
            return alive, new, it + 1

        def fix_cond(carry):
            prev, alive, it = carry
            changed = jnp.sum(jnp.abs(alive - prev)) > 0.0
            return changed & (it < 129)

        _, alive8, _ = jax.lax.while_loop(
            fix_cond, fix_body, (cand8 - 2.0, cand8, jnp.int32(0)))
        alive = alive8[0:1]
        kept_rows.append(alive)
        kept_cols.append(_row_to_col(alive))

    return jnp.concatenate(kept_rows, axis=0)


def _body(fg_ref, d_ref, a_ref, out_ref):
    dy = d_ref[:, 0] * _STD[0]
    dx = d_ref[:, 1] * _STD[1]
    dh = d_ref[:, 2] * _STD[2]
    dw = d_ref[:, 3] * _STD[3]
    ay1, ax1, ay2, ax2 = (a_ref[:, c] for c in range(4))
    height = ay2 - ay1
    width = ax2 - ax1
    center_y = ay1 + 0.5 * height
    center_x = ax1 + 0.5 * width
    center_y = center_y + dy * height
    center_x = center_x + dx * width
    height = height * jnp.exp(dh)
    width = width * jnp.exp(dw)
    y1 = center_y - 0.5 * height
    x1 = center_x - 0.5 * width
    y2 = y1 + height
    x2 = x1 + width
    y1 = jnp.clip(y1, 0.0, 1.0)
    x1 = jnp.clip(x1, 0.0, 1.0)
    y2 = jnp.clip(y2, 0.0, 1.0)
    x2 = jnp.clip(x2, 0.0, 1.0)

    s = fg_ref[...]
    gidx = (128 * jax.lax.broadcasted_iota(jnp.int32, (_NP // 128, 128), 0)
            + jax.lax.broadcasted_iota(jnp.int32, (_NP // 128, 128), 1)
            ).astype(jnp.float32)
    gidx = jnp.broadcast_to(gidx[None], (_B, _NP // 128, 128))

    blk = lambda x: x.reshape(_B * _NB, _BR, 128)
    key, tie, pay = _sort_blocks(blk(s), blk(gidx),
                                 [blk(y1), blk(x1), blk(y2), blk(x2)])
    ub = lambda x: x.reshape(_B, _NB, _BR, 128)
    key, tie, pay = ub(key), ub(tie), [ub(a) for a in pay]
    while key.shape[1] > 1:
        key, tie, pay = _merge_level(key, tie, pay)

    keeps = []
    for b in range(_B):
        keeps.append(_nms_keep(*(a[b, 0] for a in pay)))
    keep = jnp.stack(keeps, axis=0)
    kept_total = jnp.sum(keep, axis=(1, 2), keepdims=True)

    pos = _ploc().astype(jnp.float32)
    ckey = jnp.where(keep > 0.5, pos, pos + float(_K))
    ckey, _, cpay = _sort_blocks(ckey, None, [a[:, 0] for a in pay],
                                 ascending=True)
    slot = pos[:, :8]
    ok = (slot < jnp.minimum(kept_total, float(_KO))).astype(jnp.float32)
    for c in range(4):
        out_ref[:, c] = cpay[c][:, :8] * ok


def kernel(scores, deltas, anchors):
    b, n, _ = scores.shape
    pad = _NP - n
    fg = jnp.pad(scores[:, :, 1], ((0, 0), (0, pad)), constant_values=-1.0)
    fg = fg.reshape(b, _NP // 128, 128)
    dpk = jnp.pad(jnp.transpose(deltas, (0, 2, 1)), ((0, 0), (0, 0), (0, pad)))
    dpk = dpk.reshape(b, 4, _NP // 128, 128)
    apk = jnp.pad(jnp.transpose(anchors, (0, 2, 1)), ((0, 0), (0, 0), (0, pad)))
    apk = apk.reshape(b, 4, _NP // 128, 128)

    out = pl.pallas_call(
        _body,
        out_shape=jax.ShapeDtypeStruct((b, 4, 8, 128), jnp.float32),
    )(fg, dpk, apk)
    props = jnp.transpose(out.reshape(b, 4, 1024), (0, 2, 1))[:, :_KO, :]
    return props


# speedup vs baseline: 1.4219x; 1.1619x over previous
import functools

import jax
import jax.numpy as jnp
from jax.experimental import pallas as pl
from jax.experimental.pallas import tpu as pltpu

_STD = (0.1, 0.1, 0.2, 0.2)
_NMS_THR = 0.7
_B = 4
_N = 20000
_NP = 20480
_NB = 10
_BR = 16
_K = 2048
_KN = 2000
_KO = 1000


def _ploc():
    r = jax.lax.broadcasted_iota(jnp.int32, (1, _BR, 128), 1)
    l = jax.lax.broadcasted_iota(jnp.int32, (1, _BR, 128), 2)
    return 128 * r + l


def _hi_mask(j):
    return (_ploc() & j) != 0


def _asc_mask(k, flip):
    m = (_ploc() & k) != 0
    return ~m if flip else m


def _lane_partner(x, j):
    lo = pltpu.roll(x, (-j) % 128, x.ndim - 1)
    hi = pltpu.roll(x, j, x.ndim - 1)
    return jnp.where(_hi_mask(j), hi, lo)


def _row_partner(x, jrow):
    lead = x.shape[:-2]
    g = _BR // (2 * jrow)
    xr = x.reshape(*lead, g, 2, jrow, 128)
    sw = jnp.concatenate([xr[..., 1:2, :, :], xr[..., 0:1, :, :]], axis=-3)
    return sw.reshape(*lead, _BR, 128)


def _cmpex(key, tie, pay, partner_fn, hi, asc):
    kp = partner_fn(key)
    if tie is None:
        better = key > kp
        tp = None
    else:
        tp = partner_fn(tie)
        better = (key > kp) | ((key == kp) & (tie < tp))
    keep_self = better ^ hi ^ asc
    nkey = jnp.where(keep_self, key, kp)
    ntie = None if tie is None else jnp.where(keep_self, tie, tp)
    npay = [jnp.where(keep_self, a, partner_fn(a)) for a in pay]
    return nkey, ntie, npay


def _stage(key, tie, pay, j, asc):
    if j >= 128:
        pf = functools.partial(_row_partner, jrow=j // 128)
    else:
        pf = functools.partial(_lane_partner, j=j)
    return _cmpex(key, tie, pay, pf, _hi_mask(j), asc)


def _sort_blocks(key, tie, pay, ascending=False):
    k = 2
    while k <= _K:
        asc = _asc_mask(k, flip=ascending)
        j = k // 2
        while j >= 1:
            key, tie, pay = _stage(key, tie, pay, j, asc)
            j //= 2
        k *= 2
    return key, tie, pay


def _bitonic_merge(key, tie, pay):
    asc = jnp.zeros((1, _BR, 128), dtype=bool)
    j = _K // 2
    while j >= 1:
        key, tie, pay = _stage(key, tie, pay, j, asc)
        j //= 2
    return key, tie, pay


def _rev_block(x):
    xr = jnp.concatenate([x[..., r:r + 1, :] for r in range(_BR - 1, -1, -1)],
                         axis=-2)
    for j in (64, 32, 16, 8, 4, 2, 1):
        xr = _lane_partner(xr, j)
    return xr


def _merge_level(key, tie, pay):
    nb = key.shape[1]
    m = nb // 2
    take = lambda x, i: x[:, :2 * m].reshape(_B, m, 2, _BR, 128)[:, :, i]
    ka, kb = take(key, 0), _rev_block(take(key, 1))
    ta, tb = take(tie, 0), _rev_block(take(tie, 1))
    better = (ka > kb) | ((ka == kb) & (ta < tb))
    wk = jnp.where(better, ka, kb)
    wt = jnp.where(better, ta, tb)
    wp = [jnp.where(better, take(a, 0), _rev_block(take(a, 1))) for a in pay]
    fl = lambda x: x.reshape(_B * m, _BR, 128)
    ufl = lambda x: x.reshape(_B, m, _BR, 128)
    wk, wt, wp = _bitonic_merge(fl(wk), fl(wt), [fl(a) for a in wp])
    wk, wt, wp = ufl(wk), ufl(wt), [ufl(a) for a in wp]
    if nb % 2:
        wk = jnp.concatenate([wk, key[:, 2 * m:]], axis=1)
        wt = jnp.concatenate([wt, tie[:, 2 * m:]], axis=1)
        wp = [jnp.concatenate([a, b[:, 2 * m:]], axis=1)
              for a, b in zip(wp, pay)]
    return wk, wt, wp


def _eye128():
    a = jax.lax.broadcasted_iota(jnp.int32, (128, 128), 0)
    b = jax.lax.broadcasted_iota(jnp.int32, (128, 128), 1)
    return (a == b).astype(jnp.float32)


def _row_to_col(v):
    return jax.lax.dot_general(_eye128(), v,
                               (((1,), (1,)), ((), ())),
                               preferred_element_type=jnp.float32)


def _nms_keep(boxes_per_img):
    nimg = len(boxes_per_img)
    coords = []
    cols = []
    for (y1, x1, y2, x2) in boxes_per_img:
        area = (y2 - y1) * (x2 - x1)
        coords.append((y1, x1, y2, x2, area))
        cols.append([jnp.concatenate([_row_to_col(c[r:r + 1])
                                      for r in range(_BR)], axis=0)
                     for c in (y1, x1, y2, x2, area)])

    sub_i = jax.lax.broadcasted_iota(jnp.int32, (128, 128), 0)
    lane_i = jax.lax.broadcasted_iota(jnp.int32, (128, 128), 1)
    tri = (sub_i < lane_i).astype(jnp.float32)

    kept_rows = [[] for _ in range(nimg)]
    kept_cols = [[] for _ in range(nimg)]
    for j in range(_BR):
        p0 = 128 * j
        if p0 + 128 <= _KN:
            valid = jnp.ones((1, 128), dtype=bool)
        else:
            nv = max(_KN - p0, 0)
            valid = jax.lax.broadcasted_iota(jnp.int32, (1, 128), 1) < nv
        cand8s = []
        mats = []
        for b in range(nimg):
            y1c, x1c, y2c, x2c, arc = cols[b]
            by1, bx1, by2, bx2 = (c[j:j + 1] for c in coords[b][:4])
            bar = coords[b][4][j:j + 1]
            if j == 0:
                sup = jnp.zeros((1, 128), dtype=bool)
            else:
                res = jnp.concatenate(kept_cols[b], axis=0)
                yy1 = jnp.maximum(y1c[:p0], by1)
                xx1 = jnp.maximum(x1c[:p0], bx1)
                yy2 = jnp.minimum(y2c[:p0], by2)
                xx2 = jnp.minimum(x2c[:p0], bx2)
                inter = (jnp.maximum(yy2 - yy1, 0.0)
                         * jnp.maximum(xx2 - xx1, 0.0))
                union = arc[:p0] + bar - inter
                iou = inter / jnp.maximum(union, 1e-8)
                hit = jnp.where((iou > _NMS_THR) & (res > 0.5), 1.0, 0.0)
                sup = jnp.max(hit, axis=0, keepdims=True) > 0.5
            cand = (valid & ~sup).astype(jnp.float32)

            yy1 = jnp.maximum(y1c[p0:p0 + 128], by1)
            xx1 = jnp.maximum(x1c[p0:p0 + 128], bx1)
            yy2 = jnp.minimum(y2c[p0:p0 + 128], by2)
            xx2 = jnp.minimum(x2c[p0:p0 + 128], bx2)
            inter = jnp.maximum(yy2 - yy1, 0.0) * jnp.maximum(xx2 - xx1, 0.0)
            union = arc[p0:p0 + 128] + bar - inter
            iou = inter / jnp.maximum(union, 1e-8)
            mat = jnp.where(iou > _NMS_THR, 1.0, 0.0) * tri
            mats.append(mat)
            cand8s.append(jnp.broadcast_to(cand, (8, 128)) + mat[0:8] * 0.0)

        candc = jnp.concatenate(cand8s, axis=0)

        def fix_body(carry):
            _, alive, it = carry
            hits = jnp.concatenate(
                [jax.lax.dot_general(alive[8 * b:8 * b + 8], mats[b],
                                     (((1,), (0,)), ((), ())),
                                     preferred_element_type=jnp.float32)
                 for b in range(nimg)], axis=0)
            new = candc * jnp.where(hits < 0.5, 1.0, 0.0)
            return alive, new, it + 1

        def fix_cond(carry):
            prev, alive, it = carry
            changed = jnp.sum(jnp.abs(alive - prev)) > 0.0
            return changed & (it < 129)

        _, alivec, _ = jax.lax.while_loop(
            fix_cond, fix_body, (candc - 2.0, candc, jnp.int32(0)))
        for b in range(nimg):
            alive = alivec[8 * b:8 * b + 1]
            kept_rows[b].append(alive)
            kept_cols[b].append(_row_to_col(alive))

    return jnp.stack([jnp.concatenate(r, axis=0) for r in kept_rows], axis=0)


def _body(fg_ref, d_ref, a_ref, out_ref):
    dy = d_ref[:, 0] * _STD[0]
    dx = d_ref[:, 1] * _STD[1]
    dh = d_ref[:, 2] * _STD[2]
    dw = d_ref[:, 3] * _STD[3]
    ay1, ax1, ay2, ax2 = (a_ref[:, c] for c in range(4))
    height = ay2 - ay1
    width = ax2 - ax1
    center_y = ay1 + 0.5 * height
    center_x = ax1 + 0.5 * width
    center_y = center_y + dy * height
    center_x = center_x + dx * width
    height = height * jnp.exp(dh)
    width = width * jnp.exp(dw)
    y1 = center_y - 0.5 * height
    x1 = center_x - 0.5 * width
    y2 = y1 + height
    x2 = x1 + width
    y1 = jnp.clip(y1, 0.0, 1.0)
    x1 = jnp.clip(x1, 0.0, 1.0)
    y2 = jnp.clip(y2, 0.0, 1.0)
    x2 = jnp.clip(x2, 0.0, 1.0)

    s = fg_ref[...]
    gidx = (128 * jax.lax.broadcasted_iota(jnp.int32, (_NP // 128, 128), 0)
            + jax.lax.broadcasted_iota(jnp.int32, (_NP // 128, 128), 1)
            ).astype(jnp.float32)
    gidx = jnp.broadcast_to(gidx[None], (_B, _NP // 128, 128))

    blk = lambda x: x.reshape(_B * _NB, _BR, 128)
    key, tie, pay = _sort_blocks(blk(s), blk(gidx),
                                 [blk(y1), blk(x1), blk(y2), blk(x2)])
    ub = lambda x: x.reshape(_B, _NB, _BR, 128)
    key, tie, pay = ub(key), ub(tie), [ub(a) for a in pay]
    while key.shape[1] > 1:
        key, tie, pay = _merge_level(key, tie, pay)

    keep = _nms_keep([tuple(a[b, 0] for a in pay) for b in range(_B)])
    kept_total = jnp.sum(keep, axis=(1, 2), keepdims=True)

    pos = _ploc().astype(jnp.float32)
    ckey = jnp.where(keep > 0.5, pos, pos + float(_K))
    ckey, _, cpay = _sort_blocks(ckey, None, [a[:, 0] for a in pay],
                                 ascending=True)
    slot = pos[:, :8]
    ok = (slot < jnp.minimum(kept_total, float(_KO))).astype(jnp.float32)
    for c in range(4):
        out_ref[:, c] = cpay[c][:, :8] * ok


def kernel(scores, deltas, anchors):
    b, n, _ = scores.shape
    pad = _NP - n
    fg = jnp.pad(scores[:, :, 1], ((0, 0), (0, pad)), constant_values=-1.0)
    fg = fg.reshape(b, _NP // 128, 128)
    dpk = jnp.pad(jnp.transpose(deltas, (0, 2, 1)), ((0, 0), (0, 0), (0, pad)))
    dpk = dpk.reshape(b, 4, _NP // 128, 128)
    apk = jnp.pad(jnp.transpose(anchors, (0, 2, 1)), ((0, 0), (0, 0), (0, pad)))
    apk = apk.reshape(b, 4, _NP // 128, 128)

    out = pl.pallas_call(
        _body,
        out_shape=jax.ShapeDtypeStruct((b, 4, 8, 128), jnp.float32),
    )(fg, dpk, apk)
    props = jnp.transpose(out.reshape(b, 4, 1024), (0, 2, 1))[:, :_KO, :]
    return props
